# Initial kernel scaffold; baseline (speedup 1.0000x reference)
import jax
import jax.numpy as jnp
from jax.experimental import pallas as pl


def kernel(x, neurons, Wq, bq, Wk, bk, Wv, bv, Wp, bp):
    raise NotImplementedError("write your pallas kernel here")


# speedup vs baseline: 7.1251x; 7.1251x over previous
import functools
import math

import jax
import jax.numpy as jnp
from jax import lax
from jax.experimental import pallas as pl
from jax.experimental.pallas import tpu as pltpu
from jax.experimental.pallas import tpu_sc as plsc

S = 2048
D = 768
NH = 12
DH = 64
NN = 4096
K = 8

BS = 256


def _bdot(a, b, dims=(((1,), (0,)), ((), ()))):
    return lax.dot_general(a.astype(jnp.bfloat16), b.astype(jnp.bfloat16),
                           dims, preferred_element_type=jnp.float32)


def _bdot_acc(a, b, dims, acc):
    (lc,), (rc,) = dims[0]
    kdim = a.shape[lc]
    out = acc
    for c in range(0, kdim, 256):
        ac = lax.slice_in_dim(a, c, min(c + 256, kdim), axis=lc)
        bc = lax.slice_in_dim(b, c, min(c + 256, kdim), axis=rc)
        out = out + lax.dot_general(
            ac.astype(jnp.bfloat16), bc.astype(jnp.bfloat16),
            dims, preferred_element_type=jnp.float32)
    return out


_KV = 1024


def _attn_body(x_ref, wq_ref, wk_ref, wv_ref, bq_ref, bk_ref, bv_ref, ctx_ref):
    x = x_ref[...]
    wq = wq_ref[0]
    wk = wk_ref[0]
    wv = wv_ref[0]
    dnT = (((0,), (1,)), ((), ()))
    q = _bdot(wq, x, dnT).T + bq_ref[0]
    k = _bdot(wk, x, dnT).T + bk_ref[0]
    v = _bdot(wv, x, dnT).T + bv_ref[0]
    m = jnp.full((S, 1), -jnp.inf, jnp.float32)
    s = jnp.zeros((S, 1), jnp.float32)
    acc = jnp.zeros((S, DH), jnp.float32)
    for j in range(S // _KV):
        kj = k[j * _KV:(j + 1) * _KV, :]
        vj = v[j * _KV:(j + 1) * _KV, :]
        a = _bdot(q, kj, (((1,), (1,)), ((), ()))) * (1.0 / math.sqrt(DH))
        mj = jnp.max(a, axis=-1, keepdims=True)
        m_new = jnp.maximum(m, mj)
        corr = jnp.where(m == m_new, 0.0, m - m_new)
        e = jnp.exp(a - m_new)
        rs = jnp.sum(e, axis=-1, keepdims=True)
        factor = jnp.exp(corr) * s
        s_new = factor + rs
        dn2 = (((1,), (0,)), ((), ()))
        q1 = _bdot(e[:, 0:256], vj[0:256, :], dn2)
        q2 = _bdot(e[:, 256:512], vj[256:512, :], dn2)
        q3 = _bdot(e[:, 512:768], vj[512:768, :], dn2)
        q4 = _bdot(e[:, 768:1024], vj[768:1024, :], dn2)
        mm = factor * acc + ((q1 + q2) + (q3 + q4))
        acc = mm * (1.0 / s_new)
        m, s = m_new, s_new
    ctx_ref[0] = acc


def _attention(x2, wq_h, wk_h, wv_h, bq_h, bk_h, bv_h):
    return pl.pallas_call(
        _attn_body,
        grid=(NH,),
        in_specs=[
            pl.BlockSpec((S, D), lambda h: (0, 0)),
            pl.BlockSpec((1, D, DH), lambda h: (h, 0, 0)),
            pl.BlockSpec((1, D, DH), lambda h: (h, 0, 0)),
            pl.BlockSpec((1, D, DH), lambda h: (h, 0, 0)),
            pl.BlockSpec((1, 1, DH), lambda h: (h, 0, 0)),
            pl.BlockSpec((1, 1, DH), lambda h: (h, 0, 0)),
            pl.BlockSpec((1, 1, DH), lambda h: (h, 0, 0)),
        ],
        out_specs=pl.BlockSpec((1, S, DH), lambda h: (h, 0, 0)),
        out_shape=jax.ShapeDtypeStruct((NH, S, DH), jnp.float32),
    )(x2, wq_h, wk_h, wv_h, bq_h, bk_h, bv_h)


def _score_body(x_ref, c_ref, n_ref, wp_ref, bp_ref, idx_ref, wts_ref):
    xb = x_ref[...]
    cb = c_ref[...]
    n = n_ref[...]
    zn = jnp.zeros((BS, NN), jnp.float32)
    dt = (((1,), (1,)), ((), ()))
    ts = _bdot_acc(xb, n, dt, zn)
    cs = _bdot_acc(cb, n, dt, zn)
    comb = jnp.concatenate([xb, cb], axis=-1)
    logit = _bdot_acc(comb, wp_ref[...], (((1,), (0,)), ((), ())),
                      jnp.zeros((BS, 2), jnp.float32)) + bp_ref[...]
    lm = jnp.max(logit, axis=-1, keepdims=True)
    le = jnp.exp(logit - lm)
    w = le / jnp.sum(le, axis=-1, keepdims=True)
    sc = w[:, 0:1] * ts + w[:, 1:2] * cs

    iota = lax.broadcasted_iota(jnp.int32, sc.shape, 1)
    idxs, vals = [], []
    cur = sc
    for _ in range(K):
        mj = jnp.max(cur, axis=-1, keepdims=True)
        ij = jnp.min(jnp.where(cur == mj, iota, NN), axis=-1, keepdims=True)
        idxs.append(ij)
        vals.append(mj)
        cur = jnp.where(iota == ij, -jnp.inf, cur)
    tidx = jnp.concatenate(idxs, axis=-1)
    tval = jnp.concatenate(vals, axis=-1)
    mv = jnp.max(tval, axis=-1, keepdims=True)
    ev = jnp.exp(tval - mv)
    idx_ref[...] = tidx
    wts_ref[...] = ev / jnp.sum(ev, axis=-1, keepdims=True)


def _score_topk(x2, context, neurons, wp, bp2):
    return pl.pallas_call(
        _score_body,
        grid=(S // BS,),
        in_specs=[
            pl.BlockSpec((BS, D), lambda i: (i, 0)),
            pl.BlockSpec((BS, D), lambda i: (i, 0)),
            pl.BlockSpec((NN, D), lambda i: (0, 0)),
            pl.BlockSpec((2 * D, 2), lambda i: (0, 0)),
            pl.BlockSpec((1, 2), lambda i: (0, 0)),
        ],
        out_specs=[
            pl.BlockSpec((BS, K), lambda i: (i, 0)),
            pl.BlockSpec((BS, K), lambda i: (i, 0)),
        ],
        out_shape=[
            jax.ShapeDtypeStruct((S, K), jnp.int32),
            jax.ShapeDtypeStruct((S, K), jnp.float32),
        ],
    )(x2, context, neurons, wp, bp2)


_NW = 32
_PER_W = (S * K) // _NW
_CH = 128
_NCHUNK = _PER_W // _CH


def _gather_body(table_hbm, idx_hbm, out_hbm, idx_v, rows_v, sem):
    wid = lax.axis_index("s") * 2 + lax.axis_index("c")
    base = wid * _PER_W
    for c in range(_NCHUNK):
        off = base + c * _CH
        pltpu.sync_copy(idx_hbm.at[pl.ds(off, _CH)], idx_v)
        pltpu.async_copy(table_hbm.at[idx_v], rows_v, sem).wait()
        pltpu.sync_copy(rows_v, out_hbm.at[pl.ds(off, _CH)])


@functools.cache
def _make_gather():
    return pl.kernel(
        _gather_body,
        out_type=jax.ShapeDtypeStruct((S * K, D), jnp.float32),
        mesh=plsc.VectorSubcoreMesh(core_axis_name="c", subcore_axis_name="s"),
        scratch_types=[
            pltpu.VMEM((_CH,), jnp.int32),
            pltpu.VMEM((_CH, D), jnp.float32),
            pltpu.SemaphoreType.DMA,
        ],
    )


def _gather(table, idx):
    return _make_gather()(table, idx)


def kernel(x, neurons, Wq, bq, Wk, bk, Wv, bv, Wp, bp):
    x2 = x.reshape(S, D)
    wq_h = Wq.reshape(D, NH, DH).transpose(1, 0, 2)
    wk_h = Wk.reshape(D, NH, DH).transpose(1, 0, 2)
    wv_h = Wv.reshape(D, NH, DH).transpose(1, 0, 2)
    bq_h = bq.reshape(NH, 1, DH)
    bk_h = bk.reshape(NH, 1, DH)
    bv_h = bv.reshape(NH, 1, DH)

    ctx_heads = _attention(x2, wq_h, wk_h, wv_h, bq_h, bk_h, bv_h)
    context = ctx_heads.transpose(1, 0, 2).reshape(S, D)

    tidx, twts = _score_topk(x2, context, neurons, Wp, bp.reshape(1, 2))

    sel = _gather(neurons, tidx.reshape(S * K))

    return (sel.reshape(1, S, K, D), tidx.reshape(1, S, K),
            twts.reshape(1, S, K), context.reshape(1, S, D))
